# baseline (speedup 1.0000x reference)
import functools

import jax
import jax.numpy as jnp
from jax.experimental import pallas as pl
from jax.experimental.pallas import tpu as pltpu

_E = 64
_NULL = 64
_K = 8
_RHO = 0.5
_IMIN = -2147483648
_FLIP = 0x7FFFFFFF
_LOW = 63


def _sortable(i):
    return jnp.where(i < 0, i ^ _FLIP, i)


def _gate_kernel(x_ref, w_ref, b_ref, n_ref, idx_ref, wgt_ref, nul_ref, aux_ref,
                 accP, accC, accS, *, n_steps, n_tokens):
    step = pl.program_id(0)

    @pl.when(step == 0)
    def _init():
        accP[...] = jnp.zeros_like(accP)
        accC[...] = jnp.zeros_like(accC)
        accS[0] = 0.0
        accS[1] = 0.0

    logits = jax.lax.dot_general(
        w_ref[...], x_ref[...], (((1,), (1,)), ((), ())),
        preferred_element_type=jnp.float32) + b_ref[...]
    nl = n_ref[0, 0]

    t_blk = logits.shape[1]
    ones_e = jnp.ones((1, _E), jnp.float32)
    ones_t = jnp.ones((t_blk, 1), jnp.float32)
    m0 = jnp.max(logits, axis=0, keepdims=True)
    er = jnp.exp(logits - m0)
    zr = jax.lax.dot_general(ones_e, er, (((1,), (0,)), ((), ())),
                             preferred_element_type=jnp.float32)
    accP[...] += jax.lax.dot_general(er * (1.0 / zr), ones_t,
                                     (((1,), (0,)), ((), ())),
                                     preferred_element_type=jnp.float32)

    M = jnp.maximum(m0, nl)
    Z = zr * jnp.exp(m0 - M) + _NULL * jnp.exp(nl - M)
    lse = M + jnp.log(Z)
    accS[0] += jnp.sum(lse * lse)

    s = _sortable(jax.lax.bitcast_convert_type(logits, jnp.int32))
    iota_e = jax.lax.broadcasted_iota(jnp.int32, logits.shape, 0)
    nkey = _sortable(jax.lax.bitcast_convert_type(
        n_ref[...], jnp.int32))[0, 0]

    null_cnt = jnp.zeros((1, logits.shape[1]), jnp.int32)
    smax_rows, am_rows, nc_rows = [], [], []
    for _ in range(_K):
        smax = jnp.max(s, axis=0, keepdims=True)
        pick = smax >= nkey
        am = jnp.min(jnp.where(s == smax, iota_e, _E), axis=0, keepdims=True)
        smax_rows.append(smax)
        am_rows.append(am)
        nc_rows.append(null_cnt)
        null_cnt = null_cnt + jnp.where(pick, 0, 1)
        am2 = jnp.where(pick, am, -1)
        s = jnp.where(iota_e == am2, _IMIN, s)

    smax_all = jnp.concatenate(smax_rows, axis=0)
    am_all = jnp.concatenate(am_rows, axis=0)
    nc_all = jnp.concatenate(nc_rows, axis=0)

    pick_all = smax_all >= nkey
    idx = jnp.where(pick_all, am_all, _E + nc_all)
    vals = jnp.where(pick_all,
                     jax.lax.bitcast_convert_type(_sortable(smax_all),
                                                  jnp.float32), nl)
    isn = jnp.where(pick_all, 0, 1)

    p_sel = jnp.exp(vals - M) * (1.0 / Z)
    rw = jnp.where(pick_all, p_sel, 0.0)
    ws = jnp.maximum(jnp.sum(rw, axis=0, keepdims=True), 1e-6)

    idx_ref[...] = idx
    wgt_ref[...] = rw * (1.0 / ws)
    nul_ref[...] = isn

    accC[...] += jax.lax.dot_general((s == _IMIN).astype(jnp.float32), ones_t,
                                     (((1,), (0,)), ((), ())),
                                     preferred_element_type=jnp.float32)
    accS[1] += jnp.sum(null_cnt.astype(jnp.float32))

    @pl.when(step == n_steps - 1)
    def _fin():
        P = accP[...] / n_tokens
        counts = accC[...]
        total = jnp.maximum(jnp.sum(counts), 1e-6)
        L_bal = _E * jnp.sum((counts / total) * P)
        null_rate = accS[1] / (n_tokens * _K)
        L_null = (null_rate - _RHO) ** 2
        L_z = accS[0] / n_tokens
        aux = 0.02 * L_bal + 0.001 * L_z + 0.01 * L_null
        aux_ref[...] = jnp.full((1, 1), aux, jnp.float32)


def kernel(x, W, logit_bias, null_logit):
    B, T, D = x.shape
    N = B * T
    xf = x.reshape(N, D)
    b = logit_bias.reshape(_E, 1).astype(jnp.float32)
    n = jnp.asarray(null_logit, jnp.float32).reshape(1, 1)

    t_blk = 2048
    n_steps = N // t_blk

    idx, wgt, nul, aux = pl.pallas_call(
        functools.partial(_gate_kernel, n_steps=n_steps, n_tokens=float(N)),
        grid=(n_steps,),
        in_specs=[
            pl.BlockSpec((t_blk, D), lambda i: (i, 0)),
            pl.BlockSpec((_E, D), lambda i: (0, 0)),
            pl.BlockSpec((_E, 1), lambda i: (0, 0)),
            pl.BlockSpec((1, 1), lambda i: (0, 0)),
        ],
        out_specs=[
            pl.BlockSpec((_K, t_blk), lambda i: (0, i)),
            pl.BlockSpec((_K, t_blk), lambda i: (0, i)),
            pl.BlockSpec((_K, t_blk), lambda i: (0, i)),
            pl.BlockSpec((1, 1), lambda i: (0, 0)),
        ],
        out_shape=[
            jax.ShapeDtypeStruct((_K, N), jnp.int32),
            jax.ShapeDtypeStruct((_K, N), jnp.float32),
            jax.ShapeDtypeStruct((_K, N), jnp.int32),
            jax.ShapeDtypeStruct((1, 1), jnp.float32),
        ],
        scratch_shapes=[
            pltpu.VMEM((_E, 1), jnp.float32),
            pltpu.VMEM((_E, 1), jnp.float32),
            pltpu.SMEM((2,), jnp.float32),
        ],
    )(xf, W, b, n)

    return (idx.T.reshape(B, T, _K),
            wgt.T.reshape(B, T, _K),
            nul.T.reshape(B, T, _K).astype(jnp.bool_),
            aux[0, 0])


# speedup vs baseline: 1.0597x; 1.0597x over previous
import functools

import jax
import jax.numpy as jnp
from jax.experimental import pallas as pl
from jax.experimental.pallas import tpu as pltpu

_E = 64
_NULL = 64
_K = 8
_RHO = 0.5
_IMIN = -2147483648
_FLIP = 0x7FFFFFFF
_LOW = 63


def _sortable(i):
    return jnp.where(i < 0, i ^ _FLIP, i)


def _gate_kernel(x_ref, w_ref, b_ref, n_ref, idx_ref, wgt_ref, nul_ref, aux_ref,
                 accP, accC, accS, *, n_steps, n_tokens):
    step = pl.program_id(0)

    @pl.when(step == 0)
    def _init():
        accP[...] = jnp.zeros_like(accP)
        accC[...] = jnp.zeros_like(accC)
        accS[0] = 0.0
        accS[1] = 0.0

    logits = jax.lax.dot_general(
        w_ref[...], x_ref[...], (((1,), (1,)), ((), ())),
        preferred_element_type=jnp.float32) + b_ref[...]
    nl = n_ref[0, 0]

    m0 = jnp.max(logits, axis=0, keepdims=True)
    er = jnp.exp(logits - m0)
    zr = jnp.sum(er, axis=0, keepdims=True)
    accP[...] += jnp.sum(er * (1.0 / zr), axis=1, keepdims=True)

    M = jnp.maximum(m0, nl)
    Z = zr * jnp.exp(m0 - M) + _NULL * jnp.exp(nl - M)
    lse = M + jnp.log(Z)
    accS[0] += jnp.sum(lse * lse)

    s = _sortable(jax.lax.bitcast_convert_type(logits, jnp.int32))
    iota_e = jax.lax.broadcasted_iota(jnp.int32, logits.shape, 0)
    nkey = _sortable(jax.lax.bitcast_convert_type(
        n_ref[...], jnp.int32))[0, 0]

    t_half = logits.shape[1] // 2
    iota_h = iota_e[:, :t_half]
    sP = [s[:, :t_half], s[:, t_half:]]
    ncP = [jnp.zeros((1, t_half), jnp.int32) for _ in range(2)]
    smaxP, amP, ncrP = ([], []), ([], []), ([], [])
    for _ in range(_K):
        for p in range(2):
            sp = sP[p]
            smax = jnp.max(sp, axis=0, keepdims=True)
            pick = smax >= nkey
            am = jnp.min(jnp.where(sp == smax, iota_h, _E), axis=0,
                         keepdims=True)
            smaxP[p].append(smax)
            amP[p].append(am)
            ncrP[p].append(ncP[p])
            ncP[p] = ncP[p] + jnp.where(pick, 0, 1)
            am2 = jnp.where(pick, am, -1)
            sP[p] = jnp.where(iota_h == am2, _IMIN, sp)

    s = jnp.concatenate([sP[0], sP[1]], axis=1)
    null_cnt = jnp.concatenate([ncP[0], ncP[1]], axis=1)
    smax_all = jnp.concatenate(
        [jnp.concatenate(smaxP[p], axis=0) for p in range(2)], axis=1)
    am_all = jnp.concatenate(
        [jnp.concatenate(amP[p], axis=0) for p in range(2)], axis=1)
    nc_all = jnp.concatenate(
        [jnp.concatenate(ncrP[p], axis=0) for p in range(2)], axis=1)

    pick_all = smax_all >= nkey
    idx = jnp.where(pick_all, am_all, _E + nc_all)
    vals = jnp.where(pick_all,
                     jax.lax.bitcast_convert_type(_sortable(smax_all),
                                                  jnp.float32), nl)
    isn = jnp.where(pick_all, 0, 1)

    p_sel = jnp.exp(vals - M) * (1.0 / Z)
    rw = jnp.where(pick_all, p_sel, 0.0)
    ws = jnp.maximum(jnp.sum(rw, axis=0, keepdims=True), 1e-6)

    idx_ref[...] = idx
    wgt_ref[...] = rw * (1.0 / ws)
    nul_ref[...] = isn

    accC[...] += jnp.sum((s == _IMIN).astype(jnp.float32), axis=1,
                         keepdims=True)
    accS[1] += jnp.sum(null_cnt.astype(jnp.float32))

    @pl.when(step == n_steps - 1)
    def _fin():
        P = accP[...] / n_tokens
        counts = accC[...]
        total = jnp.maximum(jnp.sum(counts), 1e-6)
        L_bal = _E * jnp.sum((counts / total) * P)
        null_rate = accS[1] / (n_tokens * _K)
        L_null = (null_rate - _RHO) ** 2
        L_z = accS[0] / n_tokens
        aux = 0.02 * L_bal + 0.001 * L_z + 0.01 * L_null
        aux_ref[...] = jnp.full((1, 1), aux, jnp.float32)


def kernel(x, W, logit_bias, null_logit):
    B, T, D = x.shape
    N = B * T
    xf = x.reshape(N, D)
    b = logit_bias.reshape(_E, 1).astype(jnp.float32)
    n = jnp.asarray(null_logit, jnp.float32).reshape(1, 1)

    t_blk = 2048
    n_steps = N // t_blk

    idx, wgt, nul, aux = pl.pallas_call(
        functools.partial(_gate_kernel, n_steps=n_steps, n_tokens=float(N)),
        grid=(n_steps,),
        in_specs=[
            pl.BlockSpec((t_blk, D), lambda i: (i, 0)),
            pl.BlockSpec((_E, D), lambda i: (0, 0)),
            pl.BlockSpec((_E, 1), lambda i: (0, 0)),
            pl.BlockSpec((1, 1), lambda i: (0, 0)),
        ],
        out_specs=[
            pl.BlockSpec((_K, t_blk), lambda i: (0, i)),
            pl.BlockSpec((_K, t_blk), lambda i: (0, i)),
            pl.BlockSpec((_K, t_blk), lambda i: (0, i)),
            pl.BlockSpec((1, 1), lambda i: (0, 0)),
        ],
        out_shape=[
            jax.ShapeDtypeStruct((_K, N), jnp.int32),
            jax.ShapeDtypeStruct((_K, N), jnp.float32),
            jax.ShapeDtypeStruct((_K, N), jnp.int32),
            jax.ShapeDtypeStruct((1, 1), jnp.float32),
        ],
        scratch_shapes=[
            pltpu.VMEM((_E, 1), jnp.float32),
            pltpu.VMEM((_E, 1), jnp.float32),
            pltpu.SMEM((2,), jnp.float32),
        ],
    )(xf, W, b, n)

    return (idx.T.reshape(B, T, _K),
            wgt.T.reshape(B, T, _K),
            nul.T.reshape(B, T, _K).astype(jnp.bool_),
            aux[0, 0])
